# baseline (speedup 1.0000x reference)
import functools

import jax
import jax.numpy as jnp
from jax import lax
from jax.experimental import pallas as pl
from jax.experimental.pallas import tpu as pltpu
from jax.experimental.pallas import tpu_sc as plsc

NC, NS, LANES = 2, 16, 16
NW = NC * NS
N = 10000
NT = 10112
E, EP = 160000, 80000
EB = 163840
EPB = 81920
CH = 128


def _relu(x):
    return jnp.maximum(x, 0.0)


_RPT = NT // NS
_WCHUNKS = [(i * CH, CH) for i in range(_RPT // CH)] + [(_RPT - _RPT % CH, _RPT % CH)]


def _zero_rows(buf, nrows):
    zeros = jnp.zeros((LANES,), jnp.float32)

    def zb(r, _):
        for j in range(8):
            buf[r, pl.ds(j * LANES, LANES)] = zeros
        return 0
    lax.fori_loop(0, nrows, zb, 0)


CH1 = 64
_NCH1B = EB // NW // CH1
_NCH1P = EPB // NW // CH1


def _k1_body(src2d, dst2d, wflat, blk2d, wpflat, acc_out,
             acc_sh, r0, r1, iS, iD, wbig, ss0, ss1):
    cid = lax.axis_index("c")
    sid = lax.axis_index("s")
    wid = cid * NS + sid
    iot = lax.iota(jnp.int32, LANES)
    ones = jnp.ones((LANES,), jnp.float32)
    zeros = jnp.zeros((LANES,), jnp.float32)
    rbufs = [r0, r1]
    sems = [ss0, ss1]

    _zero_rows(r0, CH1)
    _zero_rows(r1, CH1)
    for (off, sz) in _WCHUNKS3:
        pltpu.sync_copy(r0.at[pl.ds(0, sz)], acc_sh.at[pl.ds(sid * _RPT + off, sz)])
    plsc.subcore_barrier()

    def fire1(q, b, two_tgt):
        pltpu.async_copy(rbufs[b], acc_sh.at[iS.at[q]], sems[b], add=True)
        if two_tgt:
            pltpu.async_copy(rbufs[b], acc_sh.at[iD.at[q]], sems[b], add=True)

    def wait1(q, b, n):
        for _ in range(n):
            pltpu.make_async_copy(rbufs[b], acc_sh.at[iS.at[q]], sems[b]).wait()

    def build1(q, b, col):
        rb = rbufs[b]

        def row_body(r, _):
            wspl = plsc.load_gather(
                wbig, [jnp.full((LANES,), q * CH1 + r, jnp.int32)])
            rb[r, pl.ds(0, LANES)] = jnp.where(
                iot == 0, wspl, jnp.where(iot == col, ones, zeros))
            return 0
        lax.fori_loop(0, CH1, row_body, 0)

    pltpu.sync_copy(src2d.at[pl.ds(wid * _NCH1B, _NCH1B)], iS)
    pltpu.sync_copy(dst2d.at[pl.ds(wid * _NCH1B, _NCH1B)], iD)
    pltpu.sync_copy(wflat.at[pl.ds(wid * _NCH1B * CH1, _NCH1B * CH1)], wbig)
    fire1(0, 0, True)
    fire1(0, 1, True)

    def b2b_pair(t, _):
        for b in (0, 1):
            q = t * 2 + b
            wait1(jnp.maximum(q - 2, 0), b, 2)
            build1(q, b, 1)
            fire1(q, b, True)
        return 0
    lax.fori_loop(0, _NCH1B // 2, b2b_pair, 0)

    wait1(_NCH1B - 2, 0, 2)
    wait1(_NCH1B - 1, 1, 2)
    pltpu.sync_copy(blk2d.at[pl.ds(wid * _NCH1P, _NCH1P)], iS.at[pl.ds(0, _NCH1P)])
    pltpu.sync_copy(wpflat.at[pl.ds(wid * _NCH1P * CH1, _NCH1P * CH1)],
                    wbig.at[pl.ds(0, _NCH1P * CH1)])
    _zero_rows(r0, CH1)
    _zero_rows(r1, CH1)
    fire1(0, 0, False)
    fire1(0, 1, False)

    def p2b_pair(t, _):
        for b in (0, 1):
            q = t * 2 + b
            wait1(jnp.maximum(q - 2, 0), b, 1)
            build1(q, b, 3)
            fire1(q, b, False)
        return 0
    lax.fori_loop(0, _NCH1P // 2, p2b_pair, 0)
    wait1(_NCH1P - 2, 0, 1)
    wait1(_NCH1P - 1, 1, 1)

    plsc.subcore_barrier()
    for (off, sz) in _WCHUNKS3:
        pltpu.sync_copy(acc_sh.at[pl.ds(sid * _RPT + off, sz)], r0.at[pl.ds(0, sz)])
        pltpu.sync_copy(r0.at[pl.ds(0, sz)],
                        acc_out.at[pl.ds(cid * NT + sid * _RPT + off, sz)])


def _k1_call(src_p, dst_p, w_p, blk_p, wp_p):
    mesh = plsc.VectorSubcoreMesh(core_axis_name="c", subcore_axis_name="s")
    f = pl.kernel(
        _k1_body,
        out_type=jax.ShapeDtypeStruct((NC * NT, 128), jnp.float32),
        mesh=mesh,
        compiler_params=pltpu.CompilerParams(needs_layout_passes=False),
        scratch_types=[
            pltpu.VMEM_SHARED((NT, 128), jnp.float32),
            pltpu.VMEM((CH1, 128), jnp.float32),
            pltpu.VMEM((CH1, 128), jnp.float32),
            pltpu.VMEM((_NCH1B, CH1), jnp.int32),
            pltpu.VMEM((_NCH1B, CH1), jnp.int32),
            pltpu.VMEM((_NCH1B * CH1,), jnp.float32),
            pltpu.SemaphoreType.DMA,
            pltpu.SemaphoreType.DMA,
        ],
    )
    return f(src_p.reshape(EB // CH1, CH1), dst_p.reshape(EB // CH1, CH1), w_p,
             blk_p.reshape(EPB // CH1, CH1), wp_p)


PT = 20096
P_PINS = 20000
CH3 = 32
_WCHUNKS3 = [(i * CH3, CH3) for i in range(_RPT // CH3)] + [(_RPT - _RPT % CH3, _RPT % CH3)]
_NCHB = EB // NW // CH3
_NCHP = EPB // NW // CH3
SEG = 40


def _k3_body(ab_hbm, bb_hbm, cp_hbm, px_hbm, src2d, dst2d, wflat,
             blk2d, pin2d, wpflat, wrowb_hbm, wrowp_hbm,
             rb_out, rp_out,
             acc_sh, bA0, bB0, bC0, bD0, bA1, bB1, bC1, bD1,
             iS, iD, wbig, wr_v, gs0, gs1, ss0, ss1):
    cid = lax.axis_index("c")
    sid = lax.axis_index("s")
    wid = cid * NS + sid
    slots = [(bA0, bB0, bC0, bD0, gs0, ss0), (bA1, bB1, bC1, bD1, gs1, ss1)]

    def zero_acc(stage):
        _zero_rows(stage, CH3)
        for (off, sz) in _WCHUNKS3:
            pltpu.sync_copy(stage.at[pl.ds(0, sz)],
                            acc_sh.at[pl.ds(sid * _RPT + off, sz)])

    def writeout(stage, out_hbm):
        for (off, sz) in _WCHUNKS3:
            pltpu.sync_copy(acc_sh.at[pl.ds(sid * _RPT + off, sz)],
                            stage.at[pl.ds(0, sz)])
            pltpu.sync_copy(stage.at[pl.ds(0, sz)],
                            out_hbm.at[pl.ds(cid * NT + sid * _RPT + off, sz)])

    def run_segment(tab1, tab2, two_dir, nch, wjs):

        def fire_gathers(q, b):
            mA, mB, mC, mD, gsem, _ = slots[b]
            pltpu.async_copy(tab1.at[iS.at[q]], mA, gsem)
            pltpu.async_copy(tab2.at[iD.at[q]], mB, gsem)
            if two_dir:
                pltpu.async_copy(tab1.at[iD.at[q]], mC, gsem)
                pltpu.async_copy(tab2.at[iS.at[q]], mD, gsem)

        def wait_gathers(q, b):
            mA, mB, mC, mD, gsem, _ = slots[b]
            pltpu.make_async_copy(tab1.at[iS.at[q]], mA, gsem).wait()
            pltpu.make_async_copy(tab2.at[iD.at[q]], mB, gsem).wait()
            if two_dir:
                pltpu.make_async_copy(tab1.at[iD.at[q]], mC, gsem).wait()
                pltpu.make_async_copy(tab2.at[iS.at[q]], mD, gsem).wait()

        def fire_scatters(q, b):
            mA, _, mC, _, _, ssem = slots[b]
            pltpu.async_copy(mA, acc_sh.at[iS.at[q]], ssem, add=True)
            if two_dir:
                pltpu.async_copy(mC, acc_sh.at[iD.at[q]], ssem, add=True)

        def wait_scatters(q, b):
            mA, _, mC, _, _, ssem = slots[b]
            pltpu.make_async_copy(mA, acc_sh.at[iS.at[q]], ssem).wait()
            if two_dir:
                pltpu.make_async_copy(mC, acc_sh.at[iD.at[q]], ssem).wait()

        def compute(q, b):
            mA, mB, mC, mD, _, _ = slots[b]

            def row_body(r, _):
                wspl = plsc.load_gather(
                    wbig, [jnp.full((LANES,), q * CH3 + r, jnp.int32)])
                for j in range(8):
                    sl = pl.ds(j * LANES, LANES)
                    t = wspl * wjs[j]
                    mA[r, sl] = jnp.maximum(mA[r, sl] + mB[r, sl] + t, 0.0)
                    if two_dir:
                        mC[r, sl] = jnp.maximum(mC[r, sl] + mD[r, sl] + t, 0.0)
                return 0
            lax.fori_loop(0, CH3, row_body, 0)

        _zero_rows(bA1, CH3)
        if two_dir:
            _zero_rows(bC1, CH3)
        pltpu.async_copy(bA1, acc_sh.at[iS.at[0]], ss1, add=True)
        if two_dir:
            pltpu.async_copy(bC1, acc_sh.at[iD.at[0]], ss1, add=True)
        fire_gathers(0, 0)

        def pair_body(t, _):
            for b in (0, 1):
                q = t * 2 + b
                wait_gathers(q, b)
                wait_scatters(jnp.maximum(q - 1, 0), 1 - b)
                fire_gathers(jnp.minimum(q + 1, nch - 1), 1 - b)
                compute(q, b)
                fire_scatters(q, b)
            return 0
        lax.fori_loop(0, nch // 2, pair_body, 0)

        wait_scatters(nch - 1, 1)
        wait_gathers(nch - 1, 0)

    def run_phase(tab1, tab2, two_dir, idxS_hbm, idxD_hbm, wf_hbm, nch_tile,
                  wrow_hbm, out_hbm):
        zero_acc(bA0)
        pltpu.sync_copy(wrow_hbm, wr_v)
        wjs = [wr_v[pl.ds(j * LANES, LANES)] for j in range(8)]
        plsc.subcore_barrier()
        for s in range(nch_tile // SEG):
            row0 = wid * nch_tile + s * SEG
            pltpu.sync_copy(idxS_hbm.at[pl.ds(row0, SEG)], iS)
            pltpu.sync_copy(idxD_hbm.at[pl.ds(row0, SEG)], iD)
            pltpu.sync_copy(wf_hbm.at[pl.ds(row0 * CH3, SEG * CH3)], wbig)
            run_segment(tab1, tab2, two_dir, SEG, wjs)
        plsc.subcore_barrier()
        writeout(bA0, out_hbm)

    run_phase(ab_hbm, bb_hbm, True, src2d, dst2d, wflat, _NCHB,
              wrowb_hbm, rb_out)

    run_phase(cp_hbm, px_hbm, False, blk2d, pin2d, wpflat, _NCHP,
              wrowp_hbm, rp_out)


def _k3_call(Ab, Bb, Cp, PXs, src_p, dst_p, w_p, blk_p, pin_p, wp_p,
             wrow_b, wrow_p):
    mesh = plsc.VectorSubcoreMesh(core_axis_name="c", subcore_axis_name="s")
    f = pl.kernel(
        _k3_body,
        out_type=(jax.ShapeDtypeStruct((NC * NT, 128), jnp.float32),
                  jax.ShapeDtypeStruct((NC * NT, 128), jnp.float32)),
        mesh=mesh,
        compiler_params=pltpu.CompilerParams(needs_layout_passes=False),
        scratch_types=[
            pltpu.VMEM_SHARED((NT, 128), jnp.float32),
            pltpu.VMEM((CH3, 128), jnp.float32),
            pltpu.VMEM((CH3, 128), jnp.float32),
            pltpu.VMEM((CH3, 128), jnp.float32),
            pltpu.VMEM((CH3, 128), jnp.float32),
            pltpu.VMEM((CH3, 128), jnp.float32),
            pltpu.VMEM((CH3, 128), jnp.float32),
            pltpu.VMEM((CH3, 128), jnp.float32),
            pltpu.VMEM((CH3, 128), jnp.float32),
            pltpu.VMEM((SEG, CH3), jnp.int32),
            pltpu.VMEM((SEG, CH3), jnp.int32),
            pltpu.VMEM((SEG * CH3,), jnp.float32),
            pltpu.VMEM((128,), jnp.float32),
            pltpu.SemaphoreType.DMA,
            pltpu.SemaphoreType.DMA,
            pltpu.SemaphoreType.DMA,
            pltpu.SemaphoreType.DMA,
        ],
    )
    return f(Ab, Bb, Cp, PXs,
             src_p.reshape(EB // CH3, CH3), dst_p.reshape(EB // CH3, CH3), w_p,
             blk_p.reshape(EPB // CH3, CH3), pin_p.reshape(EPB // CH3, CH3),
             wp_p, wrow_b, wrow_p)


_BLK = 2000


def _k2a1_body(area_ref, accs_ref, w2d_ref, wp2d_ref, wrowb_ref, wrowp_ref,
               scal_ref, wrbs_ref, wrps_ref):
    total_area = jnp.maximum(jnp.sum(area_ref[...]), 1e-6)
    case_scale = jnp.maximum(jnp.sqrt(total_area), 1e-6)
    max_degree = jnp.maximum(jnp.max(jnp.abs(accs_ref[:, 0:1])), 1.0)
    max_b2b = jnp.maximum(jnp.max(jnp.abs(w2d_ref[...])), 1.0)
    max_p2b = jnp.maximum(jnp.max(jnp.abs(wp2d_ref[...])), 1.0)
    scal = jnp.stack([1.0 / case_scale, 1.0 / max_b2b, 1.0 / max_p2b,
                      1.0 / total_area, 1.0 / max_degree, 0.0, 0.0, 0.0])
    scal_ref[...] = scal[None, :]
    wrbs_ref[...] = wrowb_ref[...] * (1.0 / max_b2b)
    wrps_ref[...] = wrowp_ref[...] * (1.0 / max_p2b)


def _k2a2_body(area_ref, con_ref, accs_ref, rid_ref, ide_ref, remb_ref,
               scal_ref, x_ref, cntb_ref, cntp_ref):
    i = pl.program_id(0)
    area = area_ref[...]
    inv_ta = scal_ref[0, 3]
    cntb_ref[...] = accs_ref[:, 1:2]
    cntp_ref[...] = accs_ref[:, 3:4]
    norm_idx = ((lax.broadcasted_iota(jnp.int32, (_BLK, 1), 0)
                 + i * _BLK).astype(jnp.float32) / float(N - 1))
    feats = jnp.concatenate([
        area * inv_ta,
        jnp.sqrt(jnp.maximum(area, 0.0) * inv_ta),
        con_ref[...],
        accs_ref[:, 0:1] * scal_ref[0, 4],
        norm_idx,
        jnp.zeros((_BLK, 7), jnp.float32),
    ], axis=-1)
    onehot = (rid_ref[...] == lax.broadcasted_iota(jnp.int32, (1, 8), 1)
              ).astype(jnp.float32)
    role_e = onehot @ remb_ref[...]
    x_ref[...] = jnp.concatenate([feats, role_e, ide_ref[...]], axis=-1)


def _k2b_body(x_ref, win1_ref, bin1_ref, win2_ref, bin2_ref,
              wb1a_ref, wb1b_ref, bb1_ref, wp1a_ref, bp1_ref,
              wself_ref, ab_ref, bbt_ref, cp_ref, s_ref):
    h1 = _relu(x_ref[...] @ win1_ref[...] + bin1_ref[...])
    h = _relu(h1 @ win2_ref[...] + bin2_ref[...])
    z = jnp.zeros((NT - N, 128), jnp.float32)
    ab_ref[0:N, :] = h @ wb1a_ref[...] + bb1_ref[...]
    ab_ref[N:NT, :] = z
    bbt_ref[0:N, :] = h @ wb1b_ref[...]
    bbt_ref[N:NT, :] = z
    cp_ref[0:N, :] = h @ wp1a_ref[...] + bp1_ref[...]
    cp_ref[N:NT, :] = z
    s_ref[...] = h @ wself_ref[...]


def _k2c_body(pins_ref, wpin_ref, scal_ref, px_ref):
    px = (pins_ref[...] * scal_ref[0, 0]) @ wpin_ref[...]
    px_ref[...] = jnp.concatenate(
        [px, jnp.zeros((PT - px.shape[0], 128), jnp.float32)])


def _k5_body(rb_ref, rp_ref, s_ref, cntb_ref, cntp_ref,
             wb2_ref, bb2_ref, wp2_ref, bp2_ref, bself_ref,
             lng_ref, lnb_ref, wg1_ref, bg1_ref, wg2_ref, bg2_ref,
             h2_ref, g_ref):
    pre = (s_ref[...] + bself_ref[...]
           + rb_ref[...] @ wb2_ref[...] + cntb_ref[...] * bb2_ref[...]
           + rp_ref[...] @ wp2_ref[...] + cntp_ref[...] * bp2_ref[...])
    m = jnp.mean(pre, axis=-1, keepdims=True)
    v = jnp.mean((pre - m) ** 2, axis=-1, keepdims=True)
    h2 = (pre - m) / jnp.sqrt(v + 1e-5) * lng_ref[...] + lnb_ref[...]
    h2_ref[...] = h2
    pooled = jnp.concatenate([jnp.mean(h2, axis=0), jnp.max(h2, axis=0)])[None, :]
    g = _relu(pooled @ wg1_ref[...] + bg1_ref[...]) @ wg2_ref[...] + bg2_ref[...]
    g_ref[...] = g[0]


def kernel(area_targets, constraints, b2b_src, b2b_dst, b2b_weight, p2b_pin,
           p2b_block, p2b_weight, pins_pos, role_ids, role_emb, idx_emb,
           W_in1, b_in1, W_in2, b_in2, Wb1, bb1, Wb2, bb2, Wp1, bp1, Wp2, bp2,
           W_self, b_self, ln_g, ln_b, Wg1, bg1, Wg2, bg2):
    pad_e = EB - E
    pad_p = EPB - EP
    junk = jnp.int32(N)
    src_p = jnp.concatenate([b2b_src.astype(jnp.int32), jnp.full((pad_e,), junk, jnp.int32)])
    dst_p = jnp.concatenate([b2b_dst.astype(jnp.int32), jnp.full((pad_e,), junk, jnp.int32)])
    w_p = jnp.concatenate([b2b_weight, jnp.zeros((pad_e,), jnp.float32)])
    blk_p = jnp.concatenate([p2b_block.astype(jnp.int32), jnp.full((pad_p,), junk, jnp.int32)])
    wp_p = jnp.concatenate([p2b_weight, jnp.zeros((pad_p,), jnp.float32)])

    acc = _k1_call(src_p, dst_p, w_p, blk_p, wp_p)

    idx_e = jnp.tile(idx_emb, (N // idx_emb.shape[0] + 1, 1))[:N]
    accs = acc[:NT] + acc[NT:]
    scal, wrow_b, wrow_p = pl.pallas_call(
        _k2a1_body,
        out_shape=(jax.ShapeDtypeStruct((1, 8), jnp.float32),
                   jax.ShapeDtypeStruct((1, 128), jnp.float32),
                   jax.ShapeDtypeStruct((1, 128), jnp.float32)),
    )(area_targets[None, :], accs, w_p.reshape(-1, 128), wp_p.reshape(-1, 128),
      Wb1[256][None, :], Wp1[130][None, :])
    nblk = N // _BLK
    x, cnt_b, cnt_p = pl.pallas_call(
        _k2a2_body,
        grid=(nblk,),
        in_specs=[
            pl.BlockSpec((_BLK, 1), lambda i: (i, 0)),
            pl.BlockSpec((_BLK, 5), lambda i: (i, 0)),
            pl.BlockSpec((_BLK, 128), lambda i: (i, 0)),
            pl.BlockSpec((_BLK, 1), lambda i: (i, 0)),
            pl.BlockSpec((_BLK, 8), lambda i: (i, 0)),
            pl.BlockSpec((8, 16), lambda i: (0, 0)),
            pl.BlockSpec((1, 8), lambda i: (0, 0)),
        ],
        out_specs=[
            pl.BlockSpec((_BLK, 40), lambda i: (i, 0)),
            pl.BlockSpec((_BLK, 1), lambda i: (i, 0)),
            pl.BlockSpec((_BLK, 1), lambda i: (i, 0)),
        ],
        out_shape=(jax.ShapeDtypeStruct((N, 40), jnp.float32),
                   jax.ShapeDtypeStruct((N, 1), jnp.float32),
                   jax.ShapeDtypeStruct((N, 1), jnp.float32)),
    )(area_targets[:, None], constraints, accs[:N],
      role_ids.astype(jnp.int32)[:, None], idx_e, role_emb, scal)

    Ab, Bb, Cp, S = pl.pallas_call(
        _k2b_body,
        out_shape=(jax.ShapeDtypeStruct((NT, 128), jnp.float32),
                   jax.ShapeDtypeStruct((NT, 128), jnp.float32),
                   jax.ShapeDtypeStruct((NT, 128), jnp.float32),
                   jax.ShapeDtypeStruct((N, 128), jnp.float32)),
    )(x, W_in1, b_in1, W_in2, b_in2, Wb1[:128], Wb1[128:256], bb1,
      Wp1[:128], bp1, W_self)
    PXs = pl.pallas_call(
        _k2c_body,
        out_shape=jax.ShapeDtypeStruct((PT, 128), jnp.float32),
    )(pins_pos, Wp1[128:130], scal)

    pin_p = jnp.concatenate([p2b_pin.astype(jnp.int32),
                             jnp.full((pad_p,), jnp.int32(pins_pos.shape[0]), jnp.int32)])
    Rb2, Rp2 = _k3_call(Ab, Bb, Cp, PXs, src_p, dst_p, w_p, blk_p, pin_p, wp_p,
                        wrow_b[0], wrow_p[0])

    Rb = (Rb2[:NT] + Rb2[NT:])[:N]
    Rp = (Rp2[:NT] + Rp2[NT:])[:N]
    h2, g = pl.pallas_call(
        _k5_body,
        out_shape=(jax.ShapeDtypeStruct((N, 128), jnp.float32),
                   jax.ShapeDtypeStruct((128,), jnp.float32)),
    )(Rb, Rp, S, cnt_b, cnt_p, Wb2, bb2, Wp2, bp2, b_self,
      ln_g, ln_b, Wg1, bg1, Wg2, bg2)
    block_mask = jnp.ones((N,), dtype=bool)
    return (h2, g, block_mask)


# speedup vs baseline: 1.0329x; 1.0329x over previous
import functools

import jax
import jax.numpy as jnp
from jax import lax
from jax.experimental import pallas as pl
from jax.experimental.pallas import tpu as pltpu
from jax.experimental.pallas import tpu_sc as plsc

NC, NS, LANES = 2, 16, 16
NW = NC * NS
N = 10000
NT = 10112
E, EP = 160000, 80000
EB = 163840
EPB = 81920
CH = 128


def _relu(x):
    return jnp.maximum(x, 0.0)


_RPT = NT // NS
_WCHUNKS = [(i * CH, CH) for i in range(_RPT // CH)] + [(_RPT - _RPT % CH, _RPT % CH)]


def _zero_rows(buf, nrows):
    zeros = jnp.zeros((LANES,), jnp.float32)

    def zb(r, _):
        for j in range(8):
            buf[r, pl.ds(j * LANES, LANES)] = zeros
        return 0
    lax.fori_loop(0, nrows, zb, 0)


CH1 = 64
_NCH1B = EB // NW // CH1
_NCH1P = EPB // NW // CH1


def _k1_body(src2d, dst2d, wflat, blk2d, wpflat, acc_out,
             acc_sh, r0, r1, iS, iD, wbig, ss0, ss1):
    cid = lax.axis_index("c")
    sid = lax.axis_index("s")
    wid = cid * NS + sid
    iot = lax.iota(jnp.int32, LANES)
    ones = jnp.ones((LANES,), jnp.float32)
    zeros = jnp.zeros((LANES,), jnp.float32)
    rbufs = [r0, r1]
    sems = [ss0, ss1]

    _zero_rows(r0, CH1)
    _zero_rows(r1, CH1)
    for (off, sz) in _WCHUNKS3:
        pltpu.sync_copy(r0.at[pl.ds(0, sz)], acc_sh.at[pl.ds(sid * _RPT + off, sz)])
    plsc.subcore_barrier()

    def fire1(q, b, two_tgt):
        pltpu.async_copy(rbufs[b], acc_sh.at[iS.at[q]], sems[b], add=True)
        if two_tgt:
            pltpu.async_copy(rbufs[b], acc_sh.at[iD.at[q]], sems[b], add=True)

    def wait1(q, b, n):
        for _ in range(n):
            pltpu.make_async_copy(rbufs[b], acc_sh.at[iS.at[q]], sems[b]).wait()

    def build1(q, b, col):
        rb = rbufs[b]

        def row_body(r, _):
            wspl = plsc.load_gather(
                wbig, [jnp.full((LANES,), q * CH1 + r, jnp.int32)])
            rb[r, pl.ds(0, LANES)] = jnp.where(
                iot == 0, wspl, jnp.where(iot == col, ones, zeros))
            return 0
        lax.fori_loop(0, CH1, row_body, 0)

    pltpu.sync_copy(src2d.at[pl.ds(wid * _NCH1B, _NCH1B)], iS)
    pltpu.sync_copy(dst2d.at[pl.ds(wid * _NCH1B, _NCH1B)], iD)
    pltpu.sync_copy(wflat.at[pl.ds(wid * _NCH1B * CH1, _NCH1B * CH1)], wbig)
    fire1(0, 0, True)
    fire1(0, 1, True)

    def b2b_pair(t, _):
        for b in (0, 1):
            q = t * 2 + b
            wait1(jnp.maximum(q - 2, 0), b, 2)
            build1(q, b, 1)
            fire1(q, b, True)
        return 0
    lax.fori_loop(0, _NCH1B // 2, b2b_pair, 0)

    wait1(_NCH1B - 2, 0, 2)
    wait1(_NCH1B - 1, 1, 2)
    pltpu.sync_copy(blk2d.at[pl.ds(wid * _NCH1P, _NCH1P)], iS.at[pl.ds(0, _NCH1P)])
    pltpu.sync_copy(wpflat.at[pl.ds(wid * _NCH1P * CH1, _NCH1P * CH1)],
                    wbig.at[pl.ds(0, _NCH1P * CH1)])
    _zero_rows(r0, CH1)
    _zero_rows(r1, CH1)
    fire1(0, 0, False)
    fire1(0, 1, False)

    def p2b_pair(t, _):
        for b in (0, 1):
            q = t * 2 + b
            wait1(jnp.maximum(q - 2, 0), b, 1)
            build1(q, b, 3)
            fire1(q, b, False)
        return 0
    lax.fori_loop(0, _NCH1P // 2, p2b_pair, 0)
    wait1(_NCH1P - 2, 0, 1)
    wait1(_NCH1P - 1, 1, 1)

    plsc.subcore_barrier()
    for (off, sz) in _WCHUNKS3:
        pltpu.sync_copy(acc_sh.at[pl.ds(sid * _RPT + off, sz)], r0.at[pl.ds(0, sz)])
        pltpu.sync_copy(r0.at[pl.ds(0, sz)],
                        acc_out.at[pl.ds(cid * NT + sid * _RPT + off, sz)])


def _k1_call(src_p, dst_p, w_p, blk_p, wp_p):
    mesh = plsc.VectorSubcoreMesh(core_axis_name="c", subcore_axis_name="s")
    f = pl.kernel(
        _k1_body,
        out_type=jax.ShapeDtypeStruct((NC * NT, 128), jnp.float32),
        mesh=mesh,
        compiler_params=pltpu.CompilerParams(needs_layout_passes=False),
        scratch_types=[
            pltpu.VMEM_SHARED((NT, 128), jnp.float32),
            pltpu.VMEM((CH1, 128), jnp.float32),
            pltpu.VMEM((CH1, 128), jnp.float32),
            pltpu.VMEM((_NCH1B, CH1), jnp.int32),
            pltpu.VMEM((_NCH1B, CH1), jnp.int32),
            pltpu.VMEM((_NCH1B * CH1,), jnp.float32),
            pltpu.SemaphoreType.DMA,
            pltpu.SemaphoreType.DMA,
        ],
    )
    return f(src_p.reshape(EB // CH1, CH1), dst_p.reshape(EB // CH1, CH1), w_p,
             blk_p.reshape(EPB // CH1, CH1), wp_p)


PT = 20096
P_PINS = 20000
CH3 = 32
_WCHUNKS3 = [(i * CH3, CH3) for i in range(_RPT // CH3)] + [(_RPT - _RPT % CH3, _RPT % CH3)]
_NCHB = EB // NW // CH3
_NCHP = EPB // NW // CH3
SEG = 40


def _k3_body(ab_hbm, bb_hbm, cp_hbm, px_hbm, src2d, dst2d, wflat,
             blk2d, pin2d, wpflat, wrowb_hbm, wrowp_hbm,
             rb_out, rp_out,
             acc_sh, bA0, bB0, bC0, bD0, bA1, bB1, bC1, bD1,
             iS, iD, wbig, wr_v, gs0, gs1, ss0, ss1):
    cid = lax.axis_index("c")
    sid = lax.axis_index("s")
    wid = cid * NS + sid
    slots = [(bA0, bB0, bC0, bD0, gs0, ss0), (bA1, bB1, bC1, bD1, gs1, ss1)]

    def zero_acc(stage):
        _zero_rows(stage, CH3)
        for (off, sz) in _WCHUNKS3:
            pltpu.sync_copy(stage.at[pl.ds(0, sz)],
                            acc_sh.at[pl.ds(sid * _RPT + off, sz)])

    def writeout(stage, out_hbm):
        for (off, sz) in _WCHUNKS3:
            pltpu.sync_copy(acc_sh.at[pl.ds(sid * _RPT + off, sz)],
                            stage.at[pl.ds(0, sz)])
            pltpu.sync_copy(stage.at[pl.ds(0, sz)],
                            out_hbm.at[pl.ds(cid * NT + sid * _RPT + off, sz)])

    def run_segment(tab1, tab2, two_dir, nch, wjs):

        def fire_gathers(q, b):
            mA, mB, mC, mD, gsem, _ = slots[b]
            pltpu.async_copy(tab1.at[iS.at[q]], mA, gsem)
            pltpu.async_copy(tab2.at[iD.at[q]], mB, gsem)
            if two_dir:
                pltpu.async_copy(tab1.at[iD.at[q]], mC, gsem)
                pltpu.async_copy(tab2.at[iS.at[q]], mD, gsem)

        def wait_gathers(q, b):
            mA, mB, mC, mD, gsem, _ = slots[b]
            pltpu.make_async_copy(tab1.at[iS.at[q]], mA, gsem).wait()
            pltpu.make_async_copy(tab2.at[iD.at[q]], mB, gsem).wait()
            if two_dir:
                pltpu.make_async_copy(tab1.at[iD.at[q]], mC, gsem).wait()
                pltpu.make_async_copy(tab2.at[iS.at[q]], mD, gsem).wait()

        def fire_scatters(q, b):
            mA, _, mC, _, _, ssem = slots[b]
            pltpu.async_copy(mA, acc_sh.at[iS.at[q]], ssem, add=True)
            if two_dir:
                pltpu.async_copy(mC, acc_sh.at[iD.at[q]], ssem, add=True)

        def wait_scatters(q, b):
            mA, _, mC, _, _, ssem = slots[b]
            pltpu.make_async_copy(mA, acc_sh.at[iS.at[q]], ssem).wait()
            if two_dir:
                pltpu.make_async_copy(mC, acc_sh.at[iD.at[q]], ssem).wait()

        def compute(q, b):
            mA, mB, mC, mD, _, _ = slots[b]

            def row_body(r, _):
                wspl = plsc.load_gather(
                    wbig, [jnp.full((LANES,), q * CH3 + r, jnp.int32)])
                for j in range(8):
                    sl = pl.ds(j * LANES, LANES)
                    t = wspl * wjs[j]
                    mA[r, sl] = jnp.maximum(mA[r, sl] + mB[r, sl] + t, 0.0)
                    if two_dir:
                        mC[r, sl] = jnp.maximum(mC[r, sl] + mD[r, sl] + t, 0.0)
                return 0
            lax.fori_loop(0, CH3, row_body, 0)

        _zero_rows(bA1, CH3)
        if two_dir:
            _zero_rows(bC1, CH3)
        pltpu.async_copy(bA1, acc_sh.at[iS.at[0]], ss1, add=True)
        if two_dir:
            pltpu.async_copy(bC1, acc_sh.at[iD.at[0]], ss1, add=True)
        fire_gathers(0, 0)

        def pair_body(t, _):
            for b in (0, 1):
                q = t * 2 + b
                wait_gathers(q, b)
                wait_scatters(jnp.maximum(q - 1, 0), 1 - b)
                fire_gathers(jnp.minimum(q + 1, nch - 1), 1 - b)
                compute(q, b)
                fire_scatters(q, b)
            return 0
        lax.fori_loop(0, nch // 2, pair_body, 0)

        wait_scatters(nch - 1, 1)
        wait_gathers(nch - 1, 0)

    def run_phase(tab1, tab2, two_dir, idxS_hbm, idxD_hbm, wf_hbm, nch_tile,
                  wrow_hbm, out_hbm):
        zero_acc(bA0)
        pltpu.sync_copy(wrow_hbm, wr_v)
        wjs = [wr_v[pl.ds(j * LANES, LANES)] for j in range(8)]
        plsc.subcore_barrier()
        for s in range(nch_tile // SEG):
            row0 = wid * nch_tile + s * SEG
            pltpu.sync_copy(idxS_hbm.at[pl.ds(row0, SEG)], iS)
            pltpu.sync_copy(idxD_hbm.at[pl.ds(row0, SEG)], iD)
            pltpu.sync_copy(wf_hbm.at[pl.ds(row0 * CH3, SEG * CH3)], wbig)
            run_segment(tab1, tab2, two_dir, SEG, wjs)
        plsc.subcore_barrier()
        writeout(bA0, out_hbm)

    run_phase(ab_hbm, bb_hbm, True, src2d, dst2d, wflat, _NCHB,
              wrowb_hbm, rb_out)

    run_phase(cp_hbm, px_hbm, False, blk2d, pin2d, wpflat, _NCHP,
              wrowp_hbm, rp_out)


def _k3_call(Ab, Bb, Cp, PXs, src_p, dst_p, w_p, blk_p, pin_p, wp_p,
             wrow_b, wrow_p):
    mesh = plsc.VectorSubcoreMesh(core_axis_name="c", subcore_axis_name="s")
    f = pl.kernel(
        _k3_body,
        out_type=(jax.ShapeDtypeStruct((NC * NT, 128), jnp.float32),
                  jax.ShapeDtypeStruct((NC * NT, 128), jnp.float32)),
        mesh=mesh,
        compiler_params=pltpu.CompilerParams(needs_layout_passes=False),
        scratch_types=[
            pltpu.VMEM_SHARED((NT, 128), jnp.float32),
            pltpu.VMEM((CH3, 128), jnp.float32),
            pltpu.VMEM((CH3, 128), jnp.float32),
            pltpu.VMEM((CH3, 128), jnp.float32),
            pltpu.VMEM((CH3, 128), jnp.float32),
            pltpu.VMEM((CH3, 128), jnp.float32),
            pltpu.VMEM((CH3, 128), jnp.float32),
            pltpu.VMEM((CH3, 128), jnp.float32),
            pltpu.VMEM((CH3, 128), jnp.float32),
            pltpu.VMEM((SEG, CH3), jnp.int32),
            pltpu.VMEM((SEG, CH3), jnp.int32),
            pltpu.VMEM((SEG * CH3,), jnp.float32),
            pltpu.VMEM((128,), jnp.float32),
            pltpu.SemaphoreType.DMA,
            pltpu.SemaphoreType.DMA,
            pltpu.SemaphoreType.DMA,
            pltpu.SemaphoreType.DMA,
        ],
    )
    return f(Ab, Bb, Cp, PXs,
             src_p.reshape(EB // CH3, CH3), dst_p.reshape(EB // CH3, CH3), w_p,
             blk_p.reshape(EPB // CH3, CH3), pin_p.reshape(EPB // CH3, CH3),
             wp_p, wrow_b, wrow_p)


_BLK = 2000


def _k2a1_body(area_ref, accs_ref, w2d_ref, wp2d_ref, scal_ref):
    total_area = jnp.maximum(jnp.sum(area_ref[...]), 1e-6)
    case_scale = jnp.maximum(jnp.sqrt(total_area), 1e-6)
    max_degree = jnp.maximum(jnp.max(jnp.abs(accs_ref[:, 0:1])), 1.0)
    max_b2b = jnp.maximum(jnp.max(jnp.abs(w2d_ref[...])), 1.0)
    max_p2b = jnp.maximum(jnp.max(jnp.abs(wp2d_ref[...])), 1.0)
    scal = jnp.stack([1.0 / case_scale, 1.0 / max_b2b, 1.0 / max_p2b,
                      1.0 / total_area, 1.0 / max_degree, 0.0, 0.0, 0.0])
    scal_ref[...] = scal[None, :]


def _k2a2_body(area_ref, con_ref, accs_ref, rid_ref, ide_ref, remb_ref,
               scal_ref, x_ref, cntb_ref, cntp_ref):
    i = pl.program_id(0)
    area = area_ref[...]
    inv_ta = scal_ref[0, 3]
    cntb_ref[...] = accs_ref[:, 1:2]
    cntp_ref[...] = accs_ref[:, 3:4]
    norm_idx = ((lax.broadcasted_iota(jnp.int32, (_BLK, 1), 0)
                 + i * _BLK).astype(jnp.float32) / float(N - 1))
    feats = jnp.concatenate([
        area * inv_ta,
        jnp.sqrt(jnp.maximum(area, 0.0) * inv_ta),
        con_ref[...],
        accs_ref[:, 0:1] * scal_ref[0, 4],
        norm_idx,
        jnp.zeros((_BLK, 7), jnp.float32),
    ], axis=-1)
    onehot = (rid_ref[...] == lax.broadcasted_iota(jnp.int32, (1, 8), 1)
              ).astype(jnp.float32)
    role_e = onehot @ remb_ref[...]
    x_ref[...] = jnp.concatenate([feats, role_e, ide_ref[...]], axis=-1)


def _k2b_body(x_ref, win1_ref, bin1_ref, win2_ref, bin2_ref,
              wb1a_ref, wb1b_ref, bb1_ref, wp1a_ref, bp1_ref,
              wself_ref, wrowb_ref, wrowp_ref, scal_ref,
              ab_ref, bbt_ref, cp_ref, s_ref, wrbs_ref, wrps_ref):
    h1 = _relu(x_ref[...] @ win1_ref[...] + bin1_ref[...])
    h = _relu(h1 @ win2_ref[...] + bin2_ref[...])
    z = jnp.zeros((NT - N, 128), jnp.float32)
    ab_ref[0:N, :] = h @ wb1a_ref[...] + bb1_ref[...]
    ab_ref[N:NT, :] = z
    bbt_ref[0:N, :] = h @ wb1b_ref[...]
    bbt_ref[N:NT, :] = z
    cp_ref[0:N, :] = h @ wp1a_ref[...] + bp1_ref[...]
    cp_ref[N:NT, :] = z
    s_ref[...] = h @ wself_ref[...]
    wrbs_ref[...] = wrowb_ref[...] * scal_ref[0, 1]
    wrps_ref[...] = wrowp_ref[...] * scal_ref[0, 2]


def _k2c_body(pins_ref, wpin_ref, scal_ref, px_ref):
    px = (pins_ref[...] * scal_ref[0, 0]) @ wpin_ref[...]
    px_ref[...] = jnp.concatenate(
        [px, jnp.zeros((PT - px.shape[0], 128), jnp.float32)])


def _k5_body(rb_ref, rp_ref, s_ref, cntb_ref, cntp_ref,
             wb2_ref, bb2_ref, wp2_ref, bp2_ref, bself_ref,
             lng_ref, lnb_ref, wg1_ref, bg1_ref, wg2_ref, bg2_ref,
             h2_ref, g_ref):
    pre = (s_ref[...] + bself_ref[...]
           + rb_ref[...] @ wb2_ref[...] + cntb_ref[...] * bb2_ref[...]
           + rp_ref[...] @ wp2_ref[...] + cntp_ref[...] * bp2_ref[...])
    m = jnp.mean(pre, axis=-1, keepdims=True)
    v = jnp.mean((pre - m) ** 2, axis=-1, keepdims=True)
    h2 = (pre - m) / jnp.sqrt(v + 1e-5) * lng_ref[...] + lnb_ref[...]
    h2_ref[...] = h2
    pooled = jnp.concatenate([jnp.mean(h2, axis=0), jnp.max(h2, axis=0)])[None, :]
    g = _relu(pooled @ wg1_ref[...] + bg1_ref[...]) @ wg2_ref[...] + bg2_ref[...]
    g_ref[...] = g[0]


def kernel(area_targets, constraints, b2b_src, b2b_dst, b2b_weight, p2b_pin,
           p2b_block, p2b_weight, pins_pos, role_ids, role_emb, idx_emb,
           W_in1, b_in1, W_in2, b_in2, Wb1, bb1, Wb2, bb2, Wp1, bp1, Wp2, bp2,
           W_self, b_self, ln_g, ln_b, Wg1, bg1, Wg2, bg2):
    pad_e = EB - E
    pad_p = EPB - EP
    junk = jnp.int32(N)
    src_p = jnp.concatenate([b2b_src.astype(jnp.int32), jnp.full((pad_e,), junk, jnp.int32)])
    dst_p = jnp.concatenate([b2b_dst.astype(jnp.int32), jnp.full((pad_e,), junk, jnp.int32)])
    w_p = jnp.concatenate([b2b_weight, jnp.zeros((pad_e,), jnp.float32)])
    blk_p = jnp.concatenate([p2b_block.astype(jnp.int32), jnp.full((pad_p,), junk, jnp.int32)])
    wp_p = jnp.concatenate([p2b_weight, jnp.zeros((pad_p,), jnp.float32)])

    acc = _k1_call(src_p, dst_p, w_p, blk_p, wp_p)

    idx_e = jnp.tile(idx_emb, (N // idx_emb.shape[0] + 1, 1))[:N]
    accs = acc[:NT] + acc[NT:]
    scal = pl.pallas_call(
        _k2a1_body,
        out_shape=jax.ShapeDtypeStruct((1, 8), jnp.float32),
    )(area_targets[None, :], accs, w_p.reshape(-1, 128), wp_p.reshape(-1, 128))
    nblk = N // _BLK
    x, cnt_b, cnt_p = pl.pallas_call(
        _k2a2_body,
        grid=(nblk,),
        in_specs=[
            pl.BlockSpec((_BLK, 1), lambda i: (i, 0)),
            pl.BlockSpec((_BLK, 5), lambda i: (i, 0)),
            pl.BlockSpec((_BLK, 128), lambda i: (i, 0)),
            pl.BlockSpec((_BLK, 1), lambda i: (i, 0)),
            pl.BlockSpec((_BLK, 8), lambda i: (i, 0)),
            pl.BlockSpec((8, 16), lambda i: (0, 0)),
            pl.BlockSpec((1, 8), lambda i: (0, 0)),
        ],
        out_specs=[
            pl.BlockSpec((_BLK, 40), lambda i: (i, 0)),
            pl.BlockSpec((_BLK, 1), lambda i: (i, 0)),
            pl.BlockSpec((_BLK, 1), lambda i: (i, 0)),
        ],
        out_shape=(jax.ShapeDtypeStruct((N, 40), jnp.float32),
                   jax.ShapeDtypeStruct((N, 1), jnp.float32),
                   jax.ShapeDtypeStruct((N, 1), jnp.float32)),
    )(area_targets[:, None], constraints, accs[:N],
      role_ids.astype(jnp.int32)[:, None], idx_e, role_emb, scal)

    Ab, Bb, Cp, S, wrow_b, wrow_p = pl.pallas_call(
        _k2b_body,
        out_shape=(jax.ShapeDtypeStruct((NT, 128), jnp.float32),
                   jax.ShapeDtypeStruct((NT, 128), jnp.float32),
                   jax.ShapeDtypeStruct((NT, 128), jnp.float32),
                   jax.ShapeDtypeStruct((N, 128), jnp.float32),
                   jax.ShapeDtypeStruct((1, 128), jnp.float32),
                   jax.ShapeDtypeStruct((1, 128), jnp.float32)),
    )(x, W_in1, b_in1, W_in2, b_in2, Wb1[:128], Wb1[128:256], bb1,
      Wp1[:128], bp1, W_self, Wb1[256][None, :], Wp1[130][None, :], scal)
    PXs = pl.pallas_call(
        _k2c_body,
        out_shape=jax.ShapeDtypeStruct((PT, 128), jnp.float32),
    )(pins_pos, Wp1[128:130], scal)

    pin_p = jnp.concatenate([p2b_pin.astype(jnp.int32),
                             jnp.full((pad_p,), jnp.int32(pins_pos.shape[0]), jnp.int32)])
    Rb2, Rp2 = _k3_call(Ab, Bb, Cp, PXs, src_p, dst_p, w_p, blk_p, pin_p, wp_p,
                        wrow_b[0], wrow_p[0])

    Rb = (Rb2[:NT] + Rb2[NT:])[:N]
    Rp = (Rp2[:NT] + Rp2[NT:])[:N]
    h2, g = pl.pallas_call(
        _k5_body,
        out_shape=(jax.ShapeDtypeStruct((N, 128), jnp.float32),
                   jax.ShapeDtypeStruct((128,), jnp.float32)),
    )(Rb, Rp, S, cnt_b, cnt_p, Wb2, bb2, Wp2, bp2, b_self,
      ln_g, ln_b, Wg1, bg1, Wg2, bg2)
    block_mask = jnp.ones((N,), dtype=bool)
    return (h2, g, block_mask)
